# Initial kernel scaffold; baseline (speedup 1.0000x reference)
import jax
import jax.numpy as jnp
from jax.experimental import pallas as pl


def kernel(data_1, data_2, positive_edges, negative_edges, target, pos_surr, neg_surr, params):
    raise NotImplementedError("write your pallas kernel here")


# speedup vs baseline: 1.0617x; 1.0617x over previous
import functools

import jax
import jax.numpy as jnp
from jax import lax
from jax.experimental import pallas as pl
from jax.experimental.pallas import tpu as pltpu
from jax.experimental.pallas import tpu_sc as plsc

N = 10000
E = 320000
FO = [38, 17, 40, 23]
NSUB = 16
NCORE = 2
CH = 128
KCH = 4
MACRO = CH * KCH
EPG = 327680
EPR = EPG // CH
NB = 5376
PR = 5000
JUNK = 5040
GPAD = 1966080
NEGC = 64
DEGC = 38
W = 128
RB = 1000
RG = N // RB


def _sc_segsum(p_tab, src2d, dstA2d, dstB2d):
    nmac = EPR // NSUB // KCH
    mesh = plsc.VectorSubcoreMesh(
        core_axis_name="c", subcore_axis_name="s", num_cores=1)
    rps = NB // NSUB
    zr = 168

    @functools.partial(
        pl.kernel,
        out_type=pltpu.HBM((2, NB, W), jnp.float32),
        mesh=mesh,
        scratch_types=[
            pltpu.VMEM((KCH, CH), jnp.int32),
            pltpu.VMEM((KCH, CH), jnp.int32),
            pltpu.VMEM((MACRO, W), jnp.float32),
            pltpu.VMEM((zr, W), jnp.float32),
            pltpu.VMEM_SHARED((NB, W), jnp.float32),
            pltpu.SemaphoreType.DMA,
        ],
    )
    def k(p_hbm, src_hbm, dstA_hbm, dstB_hbm, out_hbm, src_v, dst_v, rows_v,
          zb_v, acc_sh, gsem):
        s = lax.axis_index("s")

        @pl.loop(0, zr)
        def _(r):
            for j in range(W // 16):
                zb_v[r, pl.ds(j * 16, 16)] = jnp.zeros((16,), jnp.float32)

        row0 = s * (EPR // NSUB)

        for p in range(2):
            dst_hbm = dstA_hbm if p == 0 else dstB_hbm

            @pl.loop(0, rps // zr)
            def _(t):
                pltpu.sync_copy(zb_v, acc_sh.at[pl.ds(s * rps + t * zr, zr)])

            plsc.subcore_barrier()

            @pl.loop(0, nmac)
            def _(m):
                r0 = row0 + m * KCH
                pltpu.sync_copy(src_hbm.at[pl.ds(r0, KCH)], src_v)
                pltpu.sync_copy(dst_hbm.at[pl.ds(r0, KCH)], dst_v)
                cps = [
                    pltpu.async_copy(p_hbm.at[src_v.at[j]],
                                     rows_v.at[pl.ds(j * CH, CH)], gsem)
                    for j in range(KCH)
                ]
                for cp in cps:
                    cp.wait()
                for j in range(KCH):
                    pltpu.sync_copy(rows_v.at[pl.ds(j * CH, CH)],
                                    acc_sh.at[dst_v.at[j]], add=True)

            plsc.subcore_barrier()
            pltpu.sync_copy(acc_sh.at[pl.ds(s * rps, rps)],
                            out_hbm.at[p, pl.ds(s * rps, rps)])
            plsc.subcore_barrier()

    return k(p_tab, src2d, dstA2d, dstB2d)


def _sc_gather(tab, idx2d):
    per_w = GPAD // (NCORE * NSUB)
    nmac = per_w // MACRO
    mesh = plsc.VectorSubcoreMesh(core_axis_name="c", subcore_axis_name="s")

    @functools.partial(
        pl.kernel,
        out_type=pltpu.HBM((GPAD, W), jnp.float32),
        mesh=mesh,
        scratch_types=[
            pltpu.VMEM((KCH, CH), jnp.int32),
            pltpu.VMEM((MACRO, W), jnp.float32),
            pltpu.SemaphoreType.DMA,
        ],
    )
    def k(tab_hbm, idx_hbm, out_hbm, idx_v, rows_v, gsem):
        c = lax.axis_index("c")
        s = lax.axis_index("s")
        wid = c * NSUB + s
        rowbase = wid * (per_w // CH)

        @pl.loop(0, nmac)
        def _(m):
            pltpu.sync_copy(idx_hbm.at[pl.ds(rowbase + m * KCH, KCH)], idx_v)
            cps = [
                pltpu.async_copy(tab_hbm.at[idx_v.at[j]],
                                 rows_v.at[pl.ds(j * CH, CH)], gsem)
                for j in range(KCH)
            ]
            for cp in cps:
                cp.wait()
            pltpu.sync_copy(
                rows_v, out_hbm.at[pl.ds(wid * per_w + m * MACRO, MACRO)])

    return k(tab, idx2d)


def _spec_for(shape):
    if len(shape) == 2 and shape[0] == N:
        return pl.BlockSpec((RB, shape[1]), lambda i: (i, 0))
    if len(shape) == 3 and shape[1] == NB:
        return pl.BlockSpec(
            (1, RB, shape[2]),
            lambda i: (i // (PR // RB), i % (PR // RB), 0))
    if len(shape) == 3 and shape[1] == N:
        return pl.BlockSpec((shape[0], RB, shape[2]), lambda i: (0, i, 0))
    return pl.BlockSpec(shape, lambda i: (0,) * len(shape))


def _tc_rows_call(fn, out_shapes, *args):
    return pl.pallas_call(
        fn,
        grid=(RG,),
        in_specs=[_spec_for(a.shape) for a in args],
        out_specs=[_spec_for(s) for s in out_shapes],
        out_shape=[jax.ShapeDtypeStruct(s, jnp.float32) for s in out_shapes],
    )(*args)


def _pack_row(pp, pn, ones_col):
    fo = pp.shape[1]
    if ones_col:
        one = jnp.ones((RB, 1), jnp.float32)
        pad = jnp.zeros((RB, NEGC - fo - 1), jnp.float32)
        return jnp.concatenate([pp, one, pad, pn, one, pad], axis=1)
    pad = jnp.zeros((RB, NEGC - fo), jnp.float32)
    return jnp.concatenate([pp, pad, pn, pad], axis=1)


def _tc_layer0(d1, d2, wlp, wrp, bp, wln, wrn, bn):
    fo = FO[0]

    def f(d1r, d2r, wlpr, wrpr, bpr, wlnr, wrnr, bnr, p_ref, sp_ref, sn_ref):
        x1 = d1r[...]
        x2 = d2r[...]
        pp = jnp.dot(x1, wlpr[...], preferred_element_type=jnp.float32)
        pn = jnp.dot(x2, wlnr[...], preferred_element_type=jnp.float32)
        p_ref[...] = _pack_row(pp, pn, ones_col=True)
        sp_ref[...] = jnp.dot(x1, wrpr[...],
                              preferred_element_type=jnp.float32) + bpr[...]
        sn_ref[...] = jnp.dot(x2, wrnr[...],
                              preferred_element_type=jnp.float32) + bnr[...]

    return _tc_rows_call(f, [(N, W), (N, fo), (N, fo)],
                         d1, d2, wlp, wrp, bp, wln, wrn, bn)


def _hpn(Spr, Snr, fp, idp, idn, spr, snr):
    hp = jax.nn.sigmoid(Spr[0, :, :fp] * idp + spr)
    hn = jax.nn.sigmoid(Snr[0, :, NEGC:NEGC + fp] * idn + snr)
    return hp, hn


def _tc_layer1(Sp, Sn, sp, sn, ws):
    fp, fo = FO[0], FO[1]

    def f(Spr, Snr, spr, snr, wlpAr, wlpBr, wrpAr, wrpBr, bpr,
          wlnAr, wlnBr, wrnAr, wrnBr, bnr, p_ref, spo_ref, sno_ref, invd_ref):
        idp = 1.0 / jnp.maximum(Spr[0, :, DEGC:DEGC + 1], 1.0)
        idn = 1.0 / jnp.maximum(Snr[0, :, NEGC + DEGC:NEGC + DEGC + 1], 1.0)
        invd_ref[0] = jnp.broadcast_to(idp, (RB, 8))
        invd_ref[1] = jnp.broadcast_to(idn, (RB, 8))
        hp, hn = _hpn(Spr[...], Snr[...], fp, idp, idn, spr[...], snr[...])
        pp = jnp.dot(hp, wlpAr[...]) + jnp.dot(hn, wlpBr[...])
        pn = jnp.dot(hn, wlnAr[...]) + jnp.dot(hp, wlnBr[...])
        p_ref[...] = _pack_row(pp, pn, ones_col=False)
        spo_ref[...] = jnp.dot(hp, wrpAr[...]) + jnp.dot(hn, wrpBr[...]) + bpr[...]
        sno_ref[...] = jnp.dot(hn, wrnAr[...]) + jnp.dot(hp, wrnBr[...]) + bnr[...]

    return _tc_rows_call(f, [(N, W), (N, fo), (N, fo), (NCORE, N, 8)],
                         Sp, Sn, sp, sn, *ws)


def _tc_layer_mid(i, Sp, Sn, sp, sn, invd, ws):
    fp, fo = FO[i - 1], FO[i]

    def f(Spr, Snr, spr, snr, invdr, wlpAr, wlpBr, wrpAr, wrpBr, bpr,
          wlnAr, wlnBr, wrnAr, wrnBr, bnr, p_ref, spo_ref, sno_ref):
        idp = invdr[0, :, 0:1]
        idn = invdr[1, :, 0:1]
        hp, hn = _hpn(Spr[...], Snr[...], fp, idp, idn, spr[...], snr[...])
        pp = jnp.dot(hp, wlpAr[...]) + jnp.dot(hn, wlpBr[...])
        pn = jnp.dot(hn, wlnAr[...]) + jnp.dot(hp, wlnBr[...])
        p_ref[...] = _pack_row(pp, pn, ones_col=False)
        spo_ref[...] = jnp.dot(hp, wrpAr[...]) + jnp.dot(hn, wrpBr[...]) + bpr[...]
        sno_ref[...] = jnp.dot(hn, wrnAr[...]) + jnp.dot(hp, wrnBr[...]) + bnr[...]

    return _tc_rows_call(f, [(N, W), (N, fo), (N, fo)],
                         Sp, Sn, sp, sn, invd, *ws)


def _tc_ztab(Sp, Sn, sp, sn, invd):
    fp = FO[3]

    def f(Spr, Snr, spr, snr, invdr, z_ref):
        idp = invdr[0, :, 0:1]
        idn = invdr[1, :, 0:1]
        hp, hn = _hpn(Spr[...], Snr[...], fp, idp, idn, spr[...], snr[...])
        z_ref[...] = jnp.concatenate(
            [hp, hn, jnp.zeros((RB, W - 2 * fp), jnp.float32)], axis=1)

    return _tc_rows_call(f, [(N, W)], Sp, Sn, sp, sn, invd)[0]


def _tc_loss(G, toh, wt, wb):
    BLK = 1000
    NSTEP = E // BLK

    def seg_spec(seg, width):
        return pl.BlockSpec((BLK, width), lambda i, s=seg: (s * NSTEP + i, 0))

    def f(g0, g1, g2, g3, g4, g5, t0, t1, t2, t3, t4, t5, wtr, wbr,
          acc_ref, out_ref):
        i = pl.program_id(0)
        P = [g0[...], g1[...], g2[...], g3[...], g4[...], g5[...]]

        def sq(a, b):
            return jnp.sum((a - b) ** 2, axis=1)

        l1 = jnp.sum(jnp.maximum(sq(P[0], P[1]) - sq(P[0], P[2]), 0.0))
        l2 = jnp.sum(jnp.maximum(sq(P[3], P[5]) - sq(P[3], P[4]), 0.0))

        wt_ = wtr[...]
        wb_ = wbr[...]

        def mm(x, w_):
            return jnp.dot(x, w_, preferred_element_type=jnp.float32)

        at0, at1, at3, at4 = mm(P[0], wt_), mm(P[1], wt_), mm(P[3], wt_), mm(P[4], wt_)
        bb1, bb2, bb4, bb5 = mm(P[1], wb_), mm(P[2], wb_), mm(P[4], wb_), mm(P[5], wb_)

        def nll(logits, tref):
            lp = jax.nn.log_softmax(logits, axis=1)
            return jnp.sum(lp * tref[...])

        sreg = (nll(at0 + bb1, t0) + nll(at3 + bb4, t1) + nll(at3 + bb5, t2)
                + nll(at4 + bb5, t3) + nll(at0 + bb2, t4) + nll(at1 + bb2, t5))

        lane = lax.broadcasted_iota(jnp.int32, (1, 128), 1)
        vec = (jnp.where(lane == 0, sreg, 0.0)
               + jnp.where(lane == 1, l1, 0.0)
               + jnp.where(lane == 2, l2, 0.0))

        @pl.when(i == 0)
        def _():
            acc_ref[...] = jnp.zeros((1, 128), jnp.float32)
            out_ref[...] = jnp.zeros((1, 128), jnp.float32)

        acc_ref[...] += vec

        @pl.when(i == NSTEP - 1)
        def _():
            tot = acc_ref[...]
            wv = (jnp.where(lane == 0, -1.0 / (6 * E), 0.0)
                  + jnp.where((lane == 1) | (lane == 2), 0.1 / E, 0.0))
            out_ref[...] = jnp.full((1, 128), jnp.sum(tot * wv), jnp.float32)

    acc, out = pl.pallas_call(
        f,
        grid=(NSTEP,),
        in_specs=[seg_spec(s, W) for s in range(6)]
        + [seg_spec(s, 3) for s in range(6)]
        + [pl.BlockSpec((W, 3), lambda i: (0, 0)),
           pl.BlockSpec((W, 3), lambda i: (0, 0))],
        out_specs=[pl.BlockSpec((1, 128), lambda i: (0, 0)),
                   pl.BlockSpec((1, 128), lambda i: (0, 0))],
        out_shape=[jax.ShapeDtypeStruct((1, 128), jnp.float32),
                   jax.ShapeDtypeStruct((1, 128), jnp.float32)],
    )(G, G, G, G, G, G, toh, toh, toh, toh, toh, toh, wt, wb)
    return out


def _layer_weights(params, prefix, i, fp):
    wl = params[f'{prefix}{i}_Wl']
    wr = params[f'{prefix}{i}_Wr']
    b = params[f'{prefix}{i}_b'].reshape(1, -1)
    return wl[:fp], wl[fp:], wr[:fp], wr[fp:], b


def _pad_edges(arr, fill):
    return jnp.concatenate(
        [arr, jnp.full((EPG - E,), fill, jnp.int32)]).reshape(EPR, CH)


def kernel(data_1, data_2, positive_edges, negative_edges, target, pos_surr,
           neg_surr, params):
    pe_s, pe_d = positive_edges[0], positive_edges[1]
    ne_s, ne_d = negative_edges[0], negative_edges[1]
    def _remap(d):
        def one(p):
            ok = (d >= p * PR) & (d < (p + 1) * PR)
            return _pad_edges(jnp.where(ok, d - p * PR, JUNK), JUNK)
        return one(0), one(1)

    srcp, srcn = _pad_edges(pe_s, 0), _pad_edges(ne_s, 0)
    dstpA, dstpB = _remap(pe_d)
    dstnA, dstnB = _remap(ne_d)
    idx2d = jnp.concatenate(
        [pe_s, pe_d, pos_surr, ne_s, ne_d, neg_surr,
         jnp.zeros((GPAD - 6 * E,), jnp.int32)]).reshape(GPAD // CH, CH)
    toh = jax.nn.one_hot(target, 3, dtype=jnp.float32)
    reg_w = params['reg_W']
    padw = jnp.zeros((W - 46, 3), jnp.float32)
    wt = jnp.concatenate([reg_w[:46], padw], axis=0)
    wb = jnp.concatenate([reg_w[46:], padw], axis=0)

    p0, sp0, sn0 = _tc_layer0(
        data_1, data_2,
        params['pos0_Wl'], params['pos0_Wr'], params['pos0_b'].reshape(1, -1),
        params['neg0_Wl'], params['neg0_Wr'], params['neg0_b'].reshape(1, -1))
    s0p = _sc_segsum(p0, srcp, dstpA, dstpB)
    s0n = _sc_segsum(p0, srcn, dstnA, dstnB)

    ws1 = _layer_weights(params, 'pos', 1, FO[0]) + _layer_weights(
        params, 'neg', 1, FO[0])
    p1, sp1, sn1, invd = _tc_layer1(s0p, s0n, sp0, sn0, ws1)
    s1p = _sc_segsum(p1, srcp, dstpA, dstpB)
    s1n = _sc_segsum(p1, srcn, dstnA, dstnB)

    ws2 = _layer_weights(params, 'pos', 2, FO[1]) + _layer_weights(
        params, 'neg', 2, FO[1])
    p2, sp2, sn2 = _tc_layer_mid(2, s1p, s1n, sp1, sn1, invd, ws2)
    s2p = _sc_segsum(p2, srcp, dstpA, dstpB)
    s2n = _sc_segsum(p2, srcn, dstnA, dstnB)

    ws3 = _layer_weights(params, 'pos', 3, FO[2]) + _layer_weights(
        params, 'neg', 3, FO[2])
    p3, sp3, sn3 = _tc_layer_mid(3, s2p, s2n, sp2, sn2, invd, ws3)
    s3p = _sc_segsum(p3, srcp, dstpA, dstpB)
    s3n = _sc_segsum(p3, srcn, dstnA, dstnB)

    ztab = _tc_ztab(s3p, s3n, sp3, sn3, invd)
    G = _sc_gather(ztab, idx2d)

    out = _tc_loss(G, toh, wt, wb)
    return jnp.reshape(out[0, 0], ())
